# baseline (speedup 1.0000x reference)
import functools
import math

import jax
import jax.numpy as jnp
from jax import lax
from jax.experimental import pallas as pl
from jax.experimental.pallas import tpu as pltpu
from jax.experimental.pallas import tpu_sc as plsc

D_MODEL = 64
SCALE = math.sqrt(D_MODEL)

NC = 2
NS = 16
NW = NC * NS
NBUF = 4
CH_OFF = (0, 64, 128, 192)
CH_SZ = (64, 64, 64, 8)


def _make_kernel(R, S):
    assert S == sum(CH_SZ) and R % NW == 0
    rpw = R // NW
    npw = rpw * S
    mesh = plsc.VectorSubcoreMesh(
        core_axis_name="c", subcore_axis_name="s", num_cores=NC,
        num_subcores=NS)

    @functools.partial(
        pl.kernel,
        out_type=jax.ShapeDtypeStruct((R, S, D_MODEL), jnp.float32),
        mesh=mesh,
        scratch_types=[
            pltpu.VMEM((npw + 16,), jnp.int32),
            pltpu.VMEM((NBUF, 64), jnp.int32),
            pltpu.VMEM((NBUF, 64, 2 * D_MODEL), jnp.float32),
            pltpu.VMEM((NBUF, 64, D_MODEL), jnp.float32),
            pltpu.SemaphoreType.DMA((NBUF,)),
            pltpu.SemaphoreType.DMA((NBUF,)),
        ],
        compiler_params=pltpu.CompilerParams(use_tc_tiling_on_sc=True),
    )
    def emb_kernel(idx_hbm, table_hbm, out_hbm, idx_v, rv_v, in_v, sc_v,
                   gsem, osem):
        wid = lax.axis_index("s") * NC + lax.axis_index("c")
        base = wid * npw
        row0 = wid * rpw
        pltpu.sync_copy(idx_hbm.at[pl.ds(base, npw)], idx_v.at[pl.ds(0, npw)])

        def fire_gather(g, b):
            f0 = g * S + CH_OFF[b]
            for c in range((CH_SZ[b] + 15) // 16):
                rv_v[b, pl.ds(c * 16, 16)] = lax.shift_right_logical(
                    idx_v[pl.ds(f0 + c * 16, 16)], 1)
            pltpu.async_copy(table_hbm.at[rv_v.at[b, pl.ds(0, CH_SZ[b])]],
                             in_v.at[b, pl.ds(0, CH_SZ[b])], gsem.at[b])

        def wait_gather(b):
            pltpu.make_async_copy(
                table_hbm.at[rv_v.at[b, pl.ds(0, CH_SZ[b])]],
                in_v.at[b, pl.ds(0, CH_SZ[b])], gsem.at[b]).wait()

        def fire_out(g, b):
            pltpu.async_copy(
                sc_v.at[b, pl.ds(0, CH_SZ[b])],
                out_hbm.at[row0 + g, pl.ds(CH_OFF[b], CH_SZ[b]), :],
                osem.at[b])

        def wait_out(b):
            pltpu.make_async_copy(
                sc_v.at[b, pl.ds(0, CH_SZ[b])],
                out_hbm.at[row0, pl.ds(CH_OFF[b], CH_SZ[b]), :],
                osem.at[b]).wait()

        def scale(g, b):
            f0 = g * S + CH_OFF[b]

            @plsc.parallel_loop(0, CH_SZ[b], unroll=4)
            def srow(k):
                v = idx_v[pl.ds(f0 + k, 16)]
                off = (v[0] & 1) * D_MODEL
                for c in range(D_MODEL // 16):
                    sc_v[b, k, pl.ds(c * 16, 16)] = (
                        in_v[b, k, pl.ds(off + c * 16, 16)] * SCALE)

        for b in range(NBUF):
            fire_gather(0, b)

        for b in range(NBUF):
            wait_gather(b)
            scale(0, b)
            fire_out(0, b)
            fire_gather(1, b)

        @pl.loop(1, rpw)
        def grp(g):
            for b in range(NBUF):
                wait_gather(b)
                wait_out(b)
                scale(g, b)
                fire_out(g, b)

                @pl.when(g < rpw - 1)
                def _():
                    fire_gather(g + 1, b)

        for b in range(NBUF):
            wait_out(b)

    return emb_kernel


def kernel(x, table):
    R, S = x.shape
    xf = x.reshape(R * S)
    table2 = table.reshape(table.shape[0] // 2, 2 * D_MODEL)
    return _make_kernel(R, S)(xf, table2)


# speedup vs baseline: 1.0870x; 1.0870x over previous
import functools
import math

import jax
import jax.numpy as jnp
from jax import lax
from jax.experimental import pallas as pl
from jax.experimental.pallas import tpu as pltpu
from jax.experimental.pallas import tpu_sc as plsc

D_MODEL = 64
SCALE = math.sqrt(D_MODEL)

NC = 2
NS = 16
NW = NC * NS

DP_SPAN = 31360
DP_CH = 320
DP_NCH = DP_SPAN // DP_CH

SB = 64
NBUF = 4


def _make_depad(V):
    assert DP_CH % 16 == 0 and DP_NCH % 2 == 0
    last = V - DP_SPAN
    mesh = plsc.VectorSubcoreMesh(
        core_axis_name="c", subcore_axis_name="s", num_cores=NC,
        num_subcores=NS)

    @functools.partial(
        pl.kernel,
        out_type=jax.ShapeDtypeStruct((V // 2, 2 * D_MODEL), jnp.float32),
        mesh=mesh,
        scratch_types=[
            pltpu.VMEM((2, DP_CH, D_MODEL), jnp.float32),
            pltpu.VMEM((2, DP_CH // 2, 2 * D_MODEL), jnp.float32),
            pltpu.SemaphoreType.DMA((2,)),
            pltpu.SemaphoreType.DMA((2,)),
        ],
        compiler_params=pltpu.CompilerParams(use_tc_tiling_on_sc=True),
    )
    def depad_kernel(table_hbm, t2_hbm, inb, outb, isem, osem):
        wid = lax.axis_index("s") * NC + lax.axis_index("c")
        base = pl.multiple_of(jnp.minimum(wid * DP_SPAN, last), 8)
        base2 = pl.multiple_of(base // 2, 8)

        def fire_in(k, s):
            off = pl.multiple_of(base + k * DP_CH, 8)
            pltpu.async_copy(table_hbm.at[pl.ds(off, DP_CH)],
                             inb.at[s], isem.at[s])

        def wait_in(s):
            pltpu.make_async_copy(table_hbm.at[pl.ds(base, DP_CH)],
                                  inb.at[s], isem.at[s]).wait()

        def fire_out(k, s):
            off = pl.multiple_of(base2 + k * (DP_CH // 2), 8)
            pltpu.async_copy(outb.at[s],
                             t2_hbm.at[pl.ds(off, DP_CH // 2)],
                             osem.at[s])

        def wait_out(s):
            pltpu.make_async_copy(outb.at[s],
                                  t2_hbm.at[pl.ds(base2, DP_CH // 2)],
                                  osem.at[s]).wait()

        def compact(s):
            @plsc.parallel_loop(0, DP_CH // 2, unroll=2)
            def pair(p):
                for j in range(2):
                    for c in range(D_MODEL // 16):
                        outb[s, p, pl.ds(j * D_MODEL + c * 16, 16)] = (
                            inb[s, 2 * p + j, pl.ds(c * 16, 16)] * SCALE)

        for s in range(2):
            fire_in(s, s)
        for s in range(2):
            wait_in(s)
            compact(s)
            fire_out(s, s)
            fire_in(2 + s, s)

        @pl.loop(1, DP_NCH // 2)
        def grp(g):
            for s in range(2):
                k = 2 * g + s
                wait_in(s)
                wait_out(s)
                compact(s)
                fire_out(k, s)

                @pl.when(k + 2 < DP_NCH)
                def _():
                    fire_in(k + 2, s)

        for s in range(2):
            wait_out(s)

    return depad_kernel


def _make_gather(B):
    assert B % (NW * SB * NBUF) == 0
    npw = B // NW
    nstep = npw // SB
    ngrp = nstep // NBUF
    mesh = plsc.VectorSubcoreMesh(
        core_axis_name="c", subcore_axis_name="s", num_cores=NC,
        num_subcores=NS)

    @functools.partial(
        pl.kernel,
        out_type=jax.ShapeDtypeStruct((B, D_MODEL), jnp.float32),
        mesh=mesh,
        scratch_types=[
            pltpu.VMEM((npw + 16,), jnp.int32),
            pltpu.VMEM((NBUF, SB), jnp.int32),
            pltpu.VMEM((NBUF, SB, 2 * D_MODEL), jnp.float32),
            pltpu.VMEM((NBUF, SB, D_MODEL), jnp.float32),
            pltpu.SemaphoreType.DMA((NBUF,)),
            pltpu.SemaphoreType.DMA((NBUF,)),
        ],
        compiler_params=pltpu.CompilerParams(use_tc_tiling_on_sc=True),
    )
    def emb_kernel(idx_hbm, table_hbm, out_hbm, idx_v, rv_v, in_v, sc_v,
                   gsem, osem):
        wid = lax.axis_index("s") * NC + lax.axis_index("c")
        base = wid * npw
        pltpu.sync_copy(idx_hbm.at[pl.ds(base, npw)], idx_v.at[pl.ds(0, npw)])

        def fire_gather(s, b):
            for c in range(SB // 16):
                rv_v[b, pl.ds(c * 16, 16)] = lax.shift_right_logical(
                    idx_v[pl.ds(s * SB + c * 16, 16)], 1)
            pltpu.async_copy(table_hbm.at[rv_v.at[b]], in_v.at[b],
                             gsem.at[b])

        def wait_gather(b):
            pltpu.make_async_copy(table_hbm.at[rv_v.at[b]], in_v.at[b],
                                  gsem.at[b]).wait()

        def fire_out(s, b):
            pltpu.async_copy(sc_v.at[b], out_hbm.at[pl.ds(base + s * SB, SB)],
                             osem.at[b])

        def wait_out(b):
            pltpu.make_async_copy(sc_v.at[b], out_hbm.at[pl.ds(base, SB)],
                                  osem.at[b]).wait()

        def select(s, b):
            @plsc.parallel_loop(0, SB, unroll=4)
            def srow(k):
                v = idx_v[pl.ds(s * SB + k, 16)]
                off = (v[0] & 1) * D_MODEL
                for c in range(D_MODEL // 16):
                    sc_v[b, k, pl.ds(c * 16, 16)] = in_v[
                        b, k, pl.ds(off + c * 16, 16)]

        for b in range(NBUF):
            fire_gather(b, b)
        for b in range(NBUF):
            wait_gather(b)
            select(b, b)
            fire_out(b, b)
            fire_gather(NBUF + b, b)

        @pl.loop(1, ngrp)
        def grp(g):
            s0 = g * NBUF
            for b in range(NBUF):
                wait_gather(b)
                wait_out(b)
                select(s0 + b, b)
                fire_out(s0 + b, b)

                @pl.when(g < ngrp - 1)
                def _():
                    fire_gather(s0 + NBUF + b, b)

        for b in range(NBUF):
            wait_out(b)

    return emb_kernel


def kernel(x, table):
    B = x.size
    xf = x.reshape(B)
    table2 = _make_depad(table.shape[0])(table)
    out = _make_gather(B)(xf, table2)
    return out.reshape(*x.shape, D_MODEL)


# speedup vs baseline: 1.0974x; 1.0095x over previous
import functools
import math

import jax
import jax.numpy as jnp
from jax import lax
from jax.experimental import pallas as pl
from jax.experimental.pallas import tpu as pltpu
from jax.experimental.pallas import tpu_sc as plsc

D_MODEL = 64
SCALE = math.sqrt(D_MODEL)

NC = 2
NS = 16
NW = NC * NS
SB = 80
NBUF = 4


def _make_kernel(B):
    assert B % (NW * SB * NBUF) == 0
    npw = B // NW
    nstep = npw // SB
    ngrp = nstep // NBUF
    mesh = plsc.VectorSubcoreMesh(
        core_axis_name="c", subcore_axis_name="s", num_cores=NC,
        num_subcores=NS)

    @functools.partial(
        pl.kernel,
        out_type=jax.ShapeDtypeStruct((B, D_MODEL), jnp.float32),
        mesh=mesh,
        scratch_types=[
            pltpu.VMEM((npw + 16,), jnp.int32),
            pltpu.VMEM((NBUF, SB), jnp.int32),
            pltpu.VMEM((NBUF, SB, 2 * D_MODEL), jnp.float32),
            pltpu.VMEM((NBUF, SB, D_MODEL), jnp.float32),
            pltpu.SemaphoreType.DMA((NBUF,)),
            pltpu.SemaphoreType.DMA((NBUF,)),
        ],
        compiler_params=pltpu.CompilerParams(use_tc_tiling_on_sc=True),
    )
    def emb_kernel(idx_hbm, table_hbm, out_hbm, idx_v, rv_v, in_v, sc_v,
                   gsem, osem):
        wid = lax.axis_index("s") * NC + lax.axis_index("c")
        base = wid * npw
        pltpu.sync_copy(idx_hbm.at[pl.ds(base, npw)], idx_v.at[pl.ds(0, npw)])

        def fire_gather(s, b):
            for c in range(SB // 16):
                rv_v[b, pl.ds(c * 16, 16)] = lax.shift_right_logical(
                    idx_v[pl.ds(s * SB + c * 16, 16)], 1)
            pltpu.async_copy(table_hbm.at[rv_v.at[b]], in_v.at[b],
                             gsem.at[b])

        def wait_gather(b):
            pltpu.make_async_copy(table_hbm.at[rv_v.at[b]], in_v.at[b],
                                  gsem.at[b]).wait()

        def fire_out(s, b):
            pltpu.async_copy(sc_v.at[b], out_hbm.at[pl.ds(base + s * SB, SB)],
                             osem.at[b])

        def wait_out(b):
            pltpu.make_async_copy(sc_v.at[b], out_hbm.at[pl.ds(base, SB)],
                                  osem.at[b]).wait()

        def scale(s, b):
            @plsc.parallel_loop(0, SB, unroll=4)
            def srow(k):
                v = idx_v[pl.ds(s * SB + k, 16)]
                off = (v[0] & 1) * D_MODEL
                for c in range(D_MODEL // 16):
                    sc_v[b, k, pl.ds(c * 16, 16)] = (
                        in_v[b, k, pl.ds(off + c * 16, 16)] * SCALE)

        for b in range(NBUF):
            fire_gather(b, b)

        for b in range(NBUF):
            wait_gather(b)
            scale(b, b)
            fire_out(b, b)
            fire_gather(NBUF + b, b)

        @pl.loop(1, ngrp)
        def grp(g):
            s0 = g * NBUF
            for b in range(NBUF):
                wait_gather(b)
                wait_out(b)
                scale(s0 + b, b)
                fire_out(s0 + b, b)

                @pl.when(g < ngrp - 1)
                def _():
                    fire_gather(s0 + NBUF + b, b)

        for b in range(NBUF):
            wait_out(b)

    return emb_kernel


def kernel(x, table):
    B = x.size
    xf = x.reshape(B)
    table2 = table.reshape(table.shape[0] // 2, 2 * D_MODEL)
    out = _make_kernel(B)(xf, table2)
    return out.reshape(*x.shape, D_MODEL)


# speedup vs baseline: 1.2212x; 1.1128x over previous
import functools
import math

import jax
import jax.numpy as jnp
from jax import lax
from jax.experimental import pallas as pl
from jax.experimental.pallas import tpu as pltpu
from jax.experimental.pallas import tpu_sc as plsc

D_MODEL = 64
SCALE = math.sqrt(D_MODEL)

NC = 2
NS = 16
NW = NC * NS

DP_SPAN = 3920
DP_CH = 40
DP_NCH = DP_SPAN // DP_CH

SB = 80
NBUF = 4


def _make_depad(NB):
    assert DP_NCH % 2 == 0
    last = NB - DP_SPAN
    mesh = plsc.VectorSubcoreMesh(
        core_axis_name="c", subcore_axis_name="s", num_cores=NC,
        num_subcores=NS)

    @functools.partial(
        pl.kernel,
        out_type=jax.ShapeDtypeStruct((NB * 4, 2 * D_MODEL), jnp.float32),
        mesh=mesh,
        scratch_types=[
            pltpu.VMEM((2, DP_CH, 8, D_MODEL), jnp.float32),
            pltpu.VMEM((2, DP_CH * 4, 2 * D_MODEL), jnp.float32),
            pltpu.SemaphoreType.DMA((2,)),
            pltpu.SemaphoreType.DMA((2,)),
        ],
        compiler_params=pltpu.CompilerParams(use_tc_tiling_on_sc=True),
    )
    def depad_kernel(table_hbm, t2_hbm, inb, outb, isem, osem):
        wid = lax.axis_index("s") * NC + lax.axis_index("c")
        base = pl.multiple_of(jnp.minimum(wid * DP_SPAN, last), 8)
        base2 = pl.multiple_of(base * 4, 8)

        def fire_in(k, s):
            off = pl.multiple_of(base + k * DP_CH, 8)
            pltpu.async_copy(table_hbm.at[pl.ds(off, DP_CH)],
                             inb.at[s], isem.at[s])

        def wait_in(s):
            pltpu.make_async_copy(table_hbm.at[pl.ds(base, DP_CH)],
                                  inb.at[s], isem.at[s]).wait()

        def fire_out(k, s):
            off = pl.multiple_of(base2 + k * (DP_CH * 4), 8)
            pltpu.async_copy(outb.at[s],
                             t2_hbm.at[pl.ds(off, DP_CH * 4)],
                             osem.at[s])

        def wait_out(s):
            pltpu.make_async_copy(outb.at[s],
                                  t2_hbm.at[pl.ds(base2, DP_CH * 4)],
                                  osem.at[s]).wait()

        def compact(s):
            @plsc.parallel_loop(0, DP_CH, unroll=2)
            def blk(q):
                for r in range(8):
                    for c in range(D_MODEL // 16):
                        outb[s, 4 * q + r // 2,
                             pl.ds((r % 2) * D_MODEL + c * 16, 16)] = (
                            inb[s, q, r, pl.ds(c * 16, 16)] * SCALE)

        for s in range(2):
            fire_in(s, s)
        for s in range(2):
            wait_in(s)
            compact(s)
            fire_out(s, s)
            fire_in(2 + s, s)

        @pl.loop(1, DP_NCH // 2)
        def grp(g):
            for s in range(2):
                k = 2 * g + s
                wait_in(s)
                wait_out(s)
                compact(s)
                fire_out(k, s)

                @pl.when(k + 2 < DP_NCH)
                def _():
                    fire_in(k + 2, s)

        for s in range(2):
            wait_out(s)

    return depad_kernel


def _make_gather(B):
    assert B % (NW * SB * NBUF) == 0
    npw = B // NW
    nstep = npw // SB
    ngrp = nstep // NBUF
    mesh = plsc.VectorSubcoreMesh(
        core_axis_name="c", subcore_axis_name="s", num_cores=NC,
        num_subcores=NS)

    @functools.partial(
        pl.kernel,
        out_type=jax.ShapeDtypeStruct((B, D_MODEL), jnp.float32),
        mesh=mesh,
        scratch_types=[
            pltpu.VMEM((npw + 16,), jnp.int32),
            pltpu.VMEM((NBUF, SB), jnp.int32),
            pltpu.VMEM((NBUF, SB, 2 * D_MODEL), jnp.float32),
            pltpu.VMEM((NBUF, SB, D_MODEL), jnp.float32),
            pltpu.SemaphoreType.DMA((NBUF,)),
            pltpu.SemaphoreType.DMA((NBUF,)),
        ],
        compiler_params=pltpu.CompilerParams(use_tc_tiling_on_sc=True),
    )
    def emb_kernel(idx_hbm, table_hbm, out_hbm, idx_v, rv_v, in_v, sc_v,
                   gsem, osem):
        wid = lax.axis_index("s") * NC + lax.axis_index("c")
        base = wid * npw
        pltpu.sync_copy(idx_hbm.at[pl.ds(base, npw)], idx_v.at[pl.ds(0, npw)])

        def fire_gather(s, b):
            for c in range(SB // 16):
                rv_v[b, pl.ds(c * 16, 16)] = lax.shift_right_logical(
                    idx_v[pl.ds(s * SB + c * 16, 16)], 1)
            pltpu.async_copy(table_hbm.at[rv_v.at[b]], in_v.at[b],
                             gsem.at[b])

        def wait_gather(b):
            pltpu.make_async_copy(table_hbm.at[rv_v.at[b]], in_v.at[b],
                                  gsem.at[b]).wait()

        def fire_out(s, b):
            pltpu.async_copy(sc_v.at[b], out_hbm.at[pl.ds(base + s * SB, SB)],
                             osem.at[b])

        def wait_out(b):
            pltpu.make_async_copy(sc_v.at[b], out_hbm.at[pl.ds(base, SB)],
                                  osem.at[b]).wait()

        def select(s, b):
            @plsc.parallel_loop(0, SB, unroll=4)
            def srow(k):
                v = idx_v[pl.ds(s * SB + k, 16)]
                off = (v[0] & 1) * D_MODEL
                for c in range(D_MODEL // 16):
                    sc_v[b, k, pl.ds(c * 16, 16)] = in_v[
                        b, k, pl.ds(off + c * 16, 16)]

        for b in range(NBUF):
            fire_gather(b, b)
        for b in range(NBUF):
            wait_gather(b)
            select(b, b)
            fire_out(b, b)
            fire_gather(NBUF + b, b)

        @pl.loop(1, ngrp)
        def grp(g):
            s0 = g * NBUF
            for b in range(NBUF):
                wait_gather(b)
                wait_out(b)
                select(s0 + b, b)
                fire_out(s0 + b, b)

                @pl.when(g < ngrp - 1)
                def _():
                    fire_gather(s0 + NBUF + b, b)

        for b in range(NBUF):
            wait_out(b)

    return emb_kernel


def kernel(x, table):
    B = x.size
    V = table.shape[0]
    xf = x.reshape(B)
    table3 = table.reshape(V // 8, 8, D_MODEL)
    table2 = _make_depad(V // 8)(table3)
    out = _make_gather(B)(xf, table2)
    return out.reshape(*x.shape, D_MODEL)


# speedup vs baseline: 1.2212x; 1.0000x over previous
import functools
import math

import jax
import jax.numpy as jnp
from jax import lax
from jax.experimental import pallas as pl
from jax.experimental.pallas import tpu as pltpu
from jax.experimental.pallas import tpu_sc as plsc

D_MODEL = 64
SCALE = math.sqrt(D_MODEL)

NC = 2
NS = 16
NW = NC * NS

DP_SPAN = 3920
DP_CH = 40
DP_NCH = DP_SPAN // DP_CH
DP_RING = 2

SB = 80
NBUF = 4


def _make_depad(NB):
    assert DP_NCH % DP_RING == 0
    last = NB - DP_SPAN
    mesh = plsc.VectorSubcoreMesh(
        core_axis_name="c", subcore_axis_name="s", num_cores=NC,
        num_subcores=NS)

    @functools.partial(
        pl.kernel,
        out_type=jax.ShapeDtypeStruct((NB * 4, 2 * D_MODEL), jnp.float32),
        mesh=mesh,
        scratch_types=[
            pltpu.VMEM((DP_RING, DP_CH, 8, D_MODEL), jnp.float32),
            pltpu.VMEM((DP_RING, DP_CH * 4, 2 * D_MODEL), jnp.float32),
            pltpu.SemaphoreType.DMA((DP_RING,)),
            pltpu.SemaphoreType.DMA((DP_RING,)),
        ],
        compiler_params=pltpu.CompilerParams(use_tc_tiling_on_sc=True),
    )
    def depad_kernel(table_hbm, t2_hbm, inb, outb, isem, osem):
        wid = lax.axis_index("s") * NC + lax.axis_index("c")
        base = pl.multiple_of(jnp.minimum(wid * DP_SPAN, last), 8)
        base2 = pl.multiple_of(base * 4, 8)

        def fire_in(k, s):
            off = pl.multiple_of(base + k * DP_CH, 8)
            pltpu.async_copy(table_hbm.at[pl.ds(off, DP_CH)],
                             inb.at[s], isem.at[s])

        def wait_in(s):
            pltpu.make_async_copy(table_hbm.at[pl.ds(base, DP_CH)],
                                  inb.at[s], isem.at[s]).wait()

        def fire_out(k, s):
            off = pl.multiple_of(base2 + k * (DP_CH * 4), 8)
            pltpu.async_copy(outb.at[s],
                             t2_hbm.at[pl.ds(off, DP_CH * 4)],
                             osem.at[s])

        def wait_out(s):
            pltpu.make_async_copy(outb.at[s],
                                  t2_hbm.at[pl.ds(base2, DP_CH * 4)],
                                  osem.at[s]).wait()

        def compact(s):
            @plsc.parallel_loop(0, DP_CH, unroll=2)
            def blk(q):
                for r in range(8):
                    for c in range(D_MODEL // 16):
                        outb[s, 4 * q + r // 2,
                             pl.ds((r % 2) * D_MODEL + c * 16, 16)] = (
                            inb[s, q, r, pl.ds(c * 16, 16)] * SCALE)

        for s in range(DP_RING):
            fire_in(s, s)
        for s in range(DP_RING):
            wait_in(s)
            compact(s)
            fire_out(s, s)
            fire_in(DP_RING + s, s)

        @pl.loop(1, DP_NCH // DP_RING)
        def grp(g):
            for s in range(DP_RING):
                k = DP_RING * g + s
                wait_in(s)
                wait_out(s)
                compact(s)
                fire_out(k, s)

                @pl.when(k + DP_RING < DP_NCH)
                def _():
                    fire_in(k + DP_RING, s)

        for s in range(DP_RING):
            wait_out(s)

    return depad_kernel


def _make_gather(B):
    assert B % (NW * SB * NBUF) == 0
    npw = B // NW
    nstep = npw // SB
    ngrp = nstep // NBUF
    mesh = plsc.VectorSubcoreMesh(
        core_axis_name="c", subcore_axis_name="s", num_cores=NC,
        num_subcores=NS)

    @functools.partial(
        pl.kernel,
        out_type=jax.ShapeDtypeStruct((B, D_MODEL), jnp.float32),
        mesh=mesh,
        scratch_types=[
            pltpu.VMEM((npw + 16,), jnp.int32),
            pltpu.VMEM((NBUF, SB), jnp.int32),
            pltpu.VMEM((NBUF, SB, 2 * D_MODEL), jnp.float32),
            pltpu.VMEM((NBUF, SB, D_MODEL), jnp.float32),
            pltpu.SemaphoreType.DMA((NBUF,)),
            pltpu.SemaphoreType.DMA((NBUF,)),
        ],
        compiler_params=pltpu.CompilerParams(use_tc_tiling_on_sc=True),
    )
    def emb_kernel(idx_hbm, table_hbm, out_hbm, idx_v, rv_v, in_v, sc_v,
                   gsem, osem):
        wid = lax.axis_index("s") * NC + lax.axis_index("c")
        base = wid * npw
        pltpu.sync_copy(idx_hbm.at[pl.ds(base, npw)], idx_v.at[pl.ds(0, npw)])

        def fire_gather(s, b):
            for c in range(SB // 16):
                rv_v[b, pl.ds(c * 16, 16)] = lax.shift_right_logical(
                    idx_v[pl.ds(s * SB + c * 16, 16)], 1)
            pltpu.async_copy(table_hbm.at[rv_v.at[b]], in_v.at[b],
                             gsem.at[b])

        def wait_gather(b):
            pltpu.make_async_copy(table_hbm.at[rv_v.at[b]], in_v.at[b],
                                  gsem.at[b]).wait()

        def fire_out(s, b):
            pltpu.async_copy(sc_v.at[b], out_hbm.at[pl.ds(base + s * SB, SB)],
                             osem.at[b])

        def wait_out(b):
            pltpu.make_async_copy(sc_v.at[b], out_hbm.at[pl.ds(base, SB)],
                                  osem.at[b]).wait()

        def select(s, b):
            @plsc.parallel_loop(0, SB, unroll=4)
            def srow(k):
                v = idx_v[pl.ds(s * SB + k, 16)]
                off = (v[0] & 1) * D_MODEL
                for c in range(D_MODEL // 16):
                    sc_v[b, k, pl.ds(c * 16, 16)] = in_v[
                        b, k, pl.ds(off + c * 16, 16)]

        for b in range(NBUF):
            fire_gather(b, b)
        for b in range(NBUF):
            wait_gather(b)
            select(b, b)
            fire_out(b, b)
            fire_gather(NBUF + b, b)

        @pl.loop(1, ngrp)
        def grp(g):
            s0 = g * NBUF
            for b in range(NBUF):
                wait_gather(b)
                wait_out(b)
                select(s0 + b, b)
                fire_out(s0 + b, b)

                @pl.when(g < ngrp - 1)
                def _():
                    fire_gather(s0 + NBUF + b, b)

        for b in range(NBUF):
            wait_out(b)

    return emb_kernel


def kernel(x, table):
    B = x.size
    V = table.shape[0]
    xf = x.reshape(B)
    table3 = table.reshape(V // 8, 8, D_MODEL)
    table2 = _make_depad(V // 8)(table3)
    out = _make_gather(B)(xf, table2)
    return out.reshape(*x.shape, D_MODEL)
